# Initial kernel scaffold; baseline (speedup 1.0000x reference)
import jax
import jax.numpy as jnp
from jax.experimental import pallas as pl


def kernel(x, pos, edge_index, W_in, W_e1, b_e1, W_e2, b_e2, W_n1, b_n1, W_out):
    raise NotImplementedError("write your pallas kernel here")


# speedup vs baseline: 4.2931x; 4.2931x over previous
import functools

import jax
import jax.numpy as jnp
from jax import lax
from jax.experimental import pallas as pl
from jax.experimental.pallas import tpu as pltpu
from jax.experimental.pallas import tpu_sc as plsc

N = 10000
E = 320000
D = 128
NC = 2
NS = 16
NW = NC * NS
EPW = E // NW
C = 80
NCHUNK = EPW // C
NPAD = 10112
RPT = NPAD // NS

f32 = jnp.float32

_mesh = plsc.VectorSubcoreMesh(
    core_axis_name="c", subcore_axis_name="s", num_cores=NC, num_subcores=NS)
_sc_params = pltpu.CompilerParams(needs_layout_passes=False)


def _prep_body(x_ref, wint_ref, wdt_ref, wst_ref, be1_ref,
               u_ref, a_ref, b_ref):
    u = jnp.dot(x_ref[...], wint_ref[...], preferred_element_type=f32)
    u_ref[...] = u
    a_ref[...] = jnp.dot(u, wdt_ref[...], preferred_element_type=f32) + be1_ref[...]
    b_ref[...] = jnp.dot(u, wst_ref[...], preferred_element_type=f32)


def _prep(x, wint, wdt, wst, be1):
    BN = 2000
    grid = (N // BN,)
    return pl.pallas_call(
        _prep_body,
        grid=grid,
        in_specs=[
            pl.BlockSpec((BN, D), lambda i: (i, 0)),
            pl.BlockSpec((D, D), lambda i: (0, 0)),
            pl.BlockSpec((D, D), lambda i: (0, 0)),
            pl.BlockSpec((D, D), lambda i: (0, 0)),
            pl.BlockSpec((1, D), lambda i: (0, 0)),
        ],
        out_specs=[
            pl.BlockSpec((BN, D), lambda i: (i, 0)),
            pl.BlockSpec((BN, D), lambda i: (i, 0)),
            pl.BlockSpec((BN, D), lambda i: (i, 0)),
        ],
        out_shape=[
            jax.ShapeDtypeStruct((N, D), f32),
            jax.ShapeDtypeStruct((N, D), f32),
            jax.ShapeDtypeStruct((N, D), f32),
        ],
    )(x, wint, wdt, wst, be1)


def _sc_gather_body(a_hbm, b_hbm, posx_hbm, posy_hbm, posz_hbm,
                    srcg_hbm, dstg_hbm,
                    arows_hbm, brows_hbm, dist_hbm,
                    idxs_v, idxd_v, posx_v, posy_v, posz_v,
                    bufA, bufB, dist_v, semA, semB):
    cid = lax.axis_index("c")
    sid = lax.axis_index("s")
    wid = sid * NC + cid
    pltpu.sync_copy(srcg_hbm.at[wid], idxs_v)
    pltpu.sync_copy(dstg_hbm.at[wid], idxd_v)
    pltpu.sync_copy(posx_hbm, posx_v)
    pltpu.sync_copy(posy_hbm, posy_v)
    pltpu.sync_copy(posz_hbm, posz_v)

    def chunk_body(j, carry):
        cp1 = pltpu.async_copy(a_hbm.at[idxd_v.at[j]], bufA, semA)
        cp2 = pltpu.async_copy(b_hbm.at[idxs_v.at[j]], bufB, semB)
        for k in range(C // 16):
            s16 = idxs_v[j, pl.ds(k * 16, 16)]
            d16 = idxd_v[j, pl.ds(k * 16, 16)]
            dx = plsc.load_gather(posx_v, [s16]) - plsc.load_gather(posx_v, [d16])
            dy = plsc.load_gather(posy_v, [s16]) - plsc.load_gather(posy_v, [d16])
            dz = plsc.load_gather(posz_v, [s16]) - plsc.load_gather(posz_v, [d16])
            dist_v[0, pl.ds(j * C + k * 16, 16)] = dx * dx + dy * dy + dz * dz
        cp1.wait()
        cp2.wait()
        base = wid * EPW + j * C
        pltpu.sync_copy(bufA, arows_hbm.at[pl.ds(base, C)])
        pltpu.sync_copy(bufB, brows_hbm.at[pl.ds(base, C)])
        return carry

    lax.fori_loop(0, NCHUNK, chunk_body, 0)
    pltpu.sync_copy(dist_v, dist_hbm.at[wid])


_sc_gather = functools.partial(
    pl.kernel,
    out_type=(
        jax.ShapeDtypeStruct((E, D), f32),
        jax.ShapeDtypeStruct((E, D), f32),
        jax.ShapeDtypeStruct((NW, 1, EPW), f32),
    ),
    mesh=_mesh,
    compiler_params=_sc_params,
    scratch_types=[
        pltpu.VMEM((NCHUNK, C), jnp.int32),
        pltpu.VMEM((NCHUNK, C), jnp.int32),
        pltpu.VMEM((N,), f32),
        pltpu.VMEM((N,), f32),
        pltpu.VMEM((N,), f32),
        pltpu.VMEM((C, D), f32),
        pltpu.VMEM((C, D), f32),
        pltpu.VMEM((1, EPW), f32),
        pltpu.SemaphoreType.DMA,
        pltpu.SemaphoreType.DMA,
    ],
)(_sc_gather_body)


def _edge_body(a_ref, b_ref, dist_ref, wd_ref, we2t_ref, be2_ref, out_ref):
    z = a_ref[...] + b_ref[...] + dist_ref[...] * wd_ref[...]
    z = z * jax.nn.sigmoid(z)
    m = jnp.dot(z, we2t_ref[...], preferred_element_type=f32) + be2_ref[...]
    out_ref[...] = m * jax.nn.sigmoid(m)


def _edge_mlp(arows, brows, dist, wd, we2t, be2):
    BE = 2000
    grid = (E // BE,)
    return pl.pallas_call(
        _edge_body,
        grid=grid,
        in_specs=[
            pl.BlockSpec((BE, D), lambda i: (i, 0)),
            pl.BlockSpec((BE, D), lambda i: (i, 0)),
            pl.BlockSpec((BE, 1), lambda i: (i, 0)),
            pl.BlockSpec((1, D), lambda i: (0, 0)),
            pl.BlockSpec((D, D), lambda i: (0, 0)),
            pl.BlockSpec((1, D), lambda i: (0, 0)),
        ],
        out_specs=pl.BlockSpec((BE, D), lambda i: (i, 0)),
        out_shape=jax.ShapeDtypeStruct((E, D), f32),
    )(arows, brows, dist, wd, we2t, be2)


def _sc_scatter_body(m2_hbm, dstg_hbm, zeros_hbm, part_hbm,
                     idxd_v, buf, agg_sh, sem):
    cid = lax.axis_index("c")
    sid = lax.axis_index("s")
    wid = sid * NC + cid
    pltpu.sync_copy(dstg_hbm.at[wid], idxd_v)
    pltpu.sync_copy(zeros_hbm.at[pl.ds(sid * RPT, RPT)],
                    agg_sh.at[pl.ds(sid * RPT, RPT)])
    plsc.subcore_barrier()

    def body(j, carry):
        pltpu.sync_copy(m2_hbm.at[pl.ds(wid * EPW + j * C, C)], buf)
        pltpu.sync_copy(buf, agg_sh.at[idxd_v.at[j]], add=True)
        return carry

    lax.fori_loop(0, NCHUNK, body, 0)
    plsc.subcore_barrier()
    pltpu.sync_copy(agg_sh.at[pl.ds(sid * RPT, RPT)],
                    part_hbm.at[cid, pl.ds(sid * RPT, RPT)])


_sc_scatter = functools.partial(
    pl.kernel,
    out_type=jax.ShapeDtypeStruct((NC, NPAD, D), f32),
    mesh=_mesh,
    compiler_params=_sc_params,
    scratch_types=[
        pltpu.VMEM((NCHUNK, C), jnp.int32),
        pltpu.VMEM((C, D), f32),
        pltpu.VMEM_SHARED((NPAD, D), f32),
        pltpu.SemaphoreType.DMA,
    ],
)(_sc_scatter_body)


def _node_body(u_ref, p0_ref, p1_ref, wn1ut_ref, wn1at_ref, bn1_ref,
               woutt_ref, y_ref):
    agg = p0_ref[...] + p1_ref[...]
    v = (jnp.dot(u_ref[...], wn1ut_ref[...], preferred_element_type=f32)
         + jnp.dot(agg, wn1at_ref[...], preferred_element_type=f32)
         + bn1_ref[...])
    y_ref[...] = jnp.dot(v, woutt_ref[...], preferred_element_type=f32)


def _node(u, p0, p1, wn1ut, wn1at, bn1, woutt):
    BN = 2000
    grid = (N // BN,)
    return pl.pallas_call(
        _node_body,
        grid=grid,
        in_specs=[
            pl.BlockSpec((BN, D), lambda i: (i, 0)),
            pl.BlockSpec((BN, D), lambda i: (i, 0)),
            pl.BlockSpec((BN, D), lambda i: (i, 0)),
            pl.BlockSpec((D, D), lambda i: (0, 0)),
            pl.BlockSpec((D, D), lambda i: (0, 0)),
            pl.BlockSpec((1, D), lambda i: (0, 0)),
            pl.BlockSpec((D, D), lambda i: (0, 0)),
        ],
        out_specs=pl.BlockSpec((BN, D), lambda i: (i, 0)),
        out_shape=jax.ShapeDtypeStruct((N, D), f32),
    )(u, p0, p1, wn1ut, wn1at, bn1, woutt)


def kernel(x, pos, edge_index, W_in, W_e1, b_e1, W_e2, b_e2, W_n1, b_n1, W_out):
    src = edge_index[0].reshape(NW, NCHUNK, C)
    dst = edge_index[1].reshape(NW, NCHUNK, C)
    posx = pos[:, 0]
    posy = pos[:, 1]
    posz = pos[:, 2]
    wd = W_e1[:, 2 * D].reshape(1, D)

    u, a, b = _prep(x, W_in.T, W_e1[:, :D].T, W_e1[:, D:2 * D].T,
                    b_e1.reshape(1, D))
    arows, brows, dist = _sc_gather(a, b, posx, posy, posz, src, dst)
    m2 = _edge_mlp(arows, brows, dist.reshape(E, 1), wd, W_e2.T,
                   b_e2.reshape(1, D))
    parts = _sc_scatter(m2, dst, jnp.zeros((NPAD, D), f32))
    y = _node(u, parts[0], parts[1], W_n1[:, :D].T, W_n1[:, D:].T,
              b_n1.reshape(1, D), W_out.T)
    return (y, pos)


# speedup vs baseline: 4.8755x; 1.1357x over previous
import functools

import jax
import jax.numpy as jnp
from jax import lax
from jax.experimental import pallas as pl
from jax.experimental.pallas import tpu as pltpu
from jax.experimental.pallas import tpu_sc as plsc

N = 10000
E = 320000
D = 128
NC = 2
NS = 16
NW = NC * NS
EPW = E // NW
C = 80
NCHUNK = EPW // C
NPAD = 10112
RPT = NPAD // NS

f32 = jnp.float32

_mesh = plsc.VectorSubcoreMesh(
    core_axis_name="c", subcore_axis_name="s", num_cores=NC, num_subcores=NS)
_sc_params = pltpu.CompilerParams(needs_layout_passes=False)


def _prep_body(x_ref, wint_ref, wdt_ref, wst_ref, be1_ref,
               u_ref, a_ref, b_ref):
    u = jnp.dot(x_ref[...], wint_ref[...], preferred_element_type=f32)
    u_ref[...] = u
    a_ref[...] = jnp.dot(u, wdt_ref[...], preferred_element_type=f32) + be1_ref[...]
    b_ref[...] = jnp.dot(u, wst_ref[...], preferred_element_type=f32)


def _prep(x, wint, wdt, wst, be1):
    BN = 2000
    grid = (N // BN,)
    return pl.pallas_call(
        _prep_body,
        grid=grid,
        in_specs=[
            pl.BlockSpec((BN, D), lambda i: (i, 0)),
            pl.BlockSpec((D, D), lambda i: (0, 0)),
            pl.BlockSpec((D, D), lambda i: (0, 0)),
            pl.BlockSpec((D, D), lambda i: (0, 0)),
            pl.BlockSpec((1, D), lambda i: (0, 0)),
        ],
        out_specs=[
            pl.BlockSpec((BN, D), lambda i: (i, 0)),
            pl.BlockSpec((BN, D), lambda i: (i, 0)),
            pl.BlockSpec((BN, D), lambda i: (i, 0)),
        ],
        out_shape=[
            jax.ShapeDtypeStruct((N, D), f32),
            jax.ShapeDtypeStruct((N, D), f32),
            jax.ShapeDtypeStruct((N, D), f32),
        ],
    )(x, wint, wdt, wst, be1)


def _sc_gather_body(a_hbm, b_hbm, posx_hbm, posy_hbm, posz_hbm,
                    srcg_hbm, dstg_hbm,
                    arows_hbm, brows_hbm, dist_hbm,
                    idxs_v, idxd_v, posx_v, posy_v, posz_v,
                    bufA0, bufB0, bufA1, bufB1, dist_v,
                    gA0, gB0, gA1, gB1, oA0, oB0, oA1, oB1):
    cid = lax.axis_index("c")
    sid = lax.axis_index("s")
    wid = sid * NC + cid
    pltpu.sync_copy(srcg_hbm.at[wid], idxs_v)
    pltpu.sync_copy(dstg_hbm.at[wid], idxd_v)
    pltpu.sync_copy(posx_hbm, posx_v)
    pltpu.sync_copy(posy_hbm, posy_v)
    pltpu.sync_copy(posz_hbm, posz_v)

    def issue_g(j, bA, bB, sA, sB):
        pltpu.async_copy(a_hbm.at[idxd_v.at[j]], bA, sA)
        pltpu.async_copy(b_hbm.at[idxs_v.at[j]], bB, sB)

    def wait_g(j, bA, bB, sA, sB):
        pltpu.make_async_copy(a_hbm.at[idxd_v.at[j]], bA, sA).wait()
        pltpu.make_async_copy(b_hbm.at[idxs_v.at[j]], bB, sB).wait()

    def issue_o(j, bA, bB, sA, sB):
        base = wid * EPW + j * C
        pltpu.async_copy(bA, arows_hbm.at[pl.ds(base, C)], sA)
        pltpu.async_copy(bB, brows_hbm.at[pl.ds(base, C)], sB)

    def wait_o(j, bA, bB, sA, sB):
        base = wid * EPW + j * C
        pltpu.make_async_copy(bA, arows_hbm.at[pl.ds(base, C)], sA).wait()
        pltpu.make_async_copy(bB, brows_hbm.at[pl.ds(base, C)], sB).wait()

    def dist_chunk(j):
        for k in range(C // 16):
            s16 = idxs_v[j, pl.ds(k * 16, 16)]
            d16 = idxd_v[j, pl.ds(k * 16, 16)]
            dx = plsc.load_gather(posx_v, [s16]) - plsc.load_gather(posx_v, [d16])
            dy = plsc.load_gather(posy_v, [s16]) - plsc.load_gather(posy_v, [d16])
            dz = plsc.load_gather(posz_v, [s16]) - plsc.load_gather(posz_v, [d16])
            dist_v[0, pl.ds(j * C + k * 16, 16)] = dx * dx + dy * dy + dz * dz

    issue_g(0, bufA0, bufB0, gA0, gB0)
    issue_g(1, bufA1, bufB1, gA1, gB1)

    def body(i, carry):
        j0 = 2 * i
        j1 = j0 + 1
        n0 = jnp.minimum(j0 + 2, NCHUNK - 1)
        n1 = jnp.minimum(j1 + 2, NCHUNK - 1)
        dist_chunk(j0)
        wait_g(j0, bufA0, bufB0, gA0, gB0)
        issue_o(j0, bufA0, bufB0, oA0, oB0)
        dist_chunk(j1)
        wait_g(j1, bufA1, bufB1, gA1, gB1)
        issue_o(j1, bufA1, bufB1, oA1, oB1)
        wait_o(j0, bufA0, bufB0, oA0, oB0)
        issue_g(n0, bufA0, bufB0, gA0, gB0)
        wait_o(j1, bufA1, bufB1, oA1, oB1)
        issue_g(n1, bufA1, bufB1, gA1, gB1)
        return carry

    lax.fori_loop(0, (NCHUNK - 1) // 2, body, 0)
    j_last = NCHUNK - 1
    dist_chunk(j_last)
    wait_g(j_last, bufA0, bufB0, gA0, gB0)
    issue_o(j_last, bufA0, bufB0, oA0, oB0)
    wait_g(j_last, bufA1, bufB1, gA1, gB1)
    issue_o(j_last, bufA1, bufB1, oA1, oB1)
    wait_o(j_last, bufA0, bufB0, oA0, oB0)
    wait_o(j_last, bufA1, bufB1, oA1, oB1)
    pltpu.sync_copy(dist_v, dist_hbm.at[wid])


_sc_gather = functools.partial(
    pl.kernel,
    out_type=(
        jax.ShapeDtypeStruct((E, D), f32),
        jax.ShapeDtypeStruct((E, D), f32),
        jax.ShapeDtypeStruct((NW, 1, EPW), f32),
    ),
    mesh=_mesh,
    compiler_params=_sc_params,
    scratch_types=[
        pltpu.VMEM((NCHUNK, C), jnp.int32),
        pltpu.VMEM((NCHUNK, C), jnp.int32),
        pltpu.VMEM((N,), f32),
        pltpu.VMEM((N,), f32),
        pltpu.VMEM((N,), f32),
        pltpu.VMEM((C, D), f32),
        pltpu.VMEM((C, D), f32),
        pltpu.VMEM((C, D), f32),
        pltpu.VMEM((C, D), f32),
        pltpu.VMEM((1, EPW), f32),
        pltpu.SemaphoreType.DMA,
        pltpu.SemaphoreType.DMA,
        pltpu.SemaphoreType.DMA,
        pltpu.SemaphoreType.DMA,
        pltpu.SemaphoreType.DMA,
        pltpu.SemaphoreType.DMA,
        pltpu.SemaphoreType.DMA,
        pltpu.SemaphoreType.DMA,
    ],
)(_sc_gather_body)


def _edge_body(a_ref, b_ref, dist_ref, wd_ref, we2t_ref, be2_ref, out_ref):
    z = a_ref[...] + b_ref[...] + dist_ref[...] * wd_ref[...]
    z = z * jax.nn.sigmoid(z)
    m = jnp.dot(z, we2t_ref[...], preferred_element_type=f32) + be2_ref[...]
    out_ref[...] = m * jax.nn.sigmoid(m)


def _edge_mlp(arows, brows, dist, wd, we2t, be2):
    BE = 2000
    grid = (E // BE,)
    return pl.pallas_call(
        _edge_body,
        grid=grid,
        in_specs=[
            pl.BlockSpec((BE, D), lambda i: (i, 0)),
            pl.BlockSpec((BE, D), lambda i: (i, 0)),
            pl.BlockSpec((BE, 1), lambda i: (i, 0)),
            pl.BlockSpec((1, D), lambda i: (0, 0)),
            pl.BlockSpec((D, D), lambda i: (0, 0)),
            pl.BlockSpec((1, D), lambda i: (0, 0)),
        ],
        out_specs=pl.BlockSpec((BE, D), lambda i: (i, 0)),
        out_shape=jax.ShapeDtypeStruct((E, D), f32),
    )(arows, brows, dist, wd, we2t, be2)


def _sc_scatter_body(m2_hbm, dstg_hbm, zeros_hbm, part_hbm,
                     idxd_v, buf0, buf1, agg_sh,
                     g0, g1, s0, s1):
    cid = lax.axis_index("c")
    sid = lax.axis_index("s")
    wid = sid * NC + cid
    pltpu.sync_copy(dstg_hbm.at[wid], idxd_v)
    pltpu.sync_copy(zeros_hbm.at[pl.ds(sid * RPT, RPT)],
                    agg_sh.at[pl.ds(sid * RPT, RPT)])
    plsc.subcore_barrier()

    def issue_g(j, b, s):
        pltpu.async_copy(m2_hbm.at[pl.ds(wid * EPW + j * C, C)], b, s)

    def wait_g(j, b, s):
        pltpu.make_async_copy(m2_hbm.at[pl.ds(wid * EPW + j * C, C)], b, s).wait()

    def issue_a(j, b, s):
        pltpu.async_copy(b, agg_sh.at[idxd_v.at[j]], s, add=True)

    def wait_a(j, b, s):
        pltpu.make_async_copy(b, agg_sh.at[idxd_v.at[j]], s).wait()

    issue_g(0, buf0, g0)
    issue_g(1, buf1, g1)

    def body(i, carry):
        j0 = 2 * i
        j1 = j0 + 1
        n0 = jnp.minimum(j0 + 2, NCHUNK - 1)
        n1 = jnp.minimum(j1 + 2, NCHUNK - 1)
        wait_g(j0, buf0, g0)
        issue_a(j0, buf0, s0)
        wait_g(j1, buf1, g1)
        issue_a(j1, buf1, s1)
        wait_a(j0, buf0, s0)
        issue_g(n0, buf0, g0)
        wait_a(j1, buf1, s1)
        issue_g(n1, buf1, g1)
        return carry

    lax.fori_loop(0, (NCHUNK - 1) // 2, body, 0)
    j_last = NCHUNK - 1
    wait_g(j_last, buf0, g0)
    issue_a(j_last, buf0, s0)
    wait_g(j_last, buf1, g1)
    wait_a(j_last, buf0, s0)
    plsc.subcore_barrier()
    pltpu.sync_copy(agg_sh.at[pl.ds(sid * RPT, RPT)],
                    part_hbm.at[cid, pl.ds(sid * RPT, RPT)])


_sc_scatter = functools.partial(
    pl.kernel,
    out_type=jax.ShapeDtypeStruct((NC, NPAD, D), f32),
    mesh=_mesh,
    compiler_params=_sc_params,
    scratch_types=[
        pltpu.VMEM((NCHUNK, C), jnp.int32),
        pltpu.VMEM((C, D), f32),
        pltpu.VMEM((C, D), f32),
        pltpu.VMEM_SHARED((NPAD, D), f32),
        pltpu.SemaphoreType.DMA,
        pltpu.SemaphoreType.DMA,
        pltpu.SemaphoreType.DMA,
        pltpu.SemaphoreType.DMA,
    ],
)(_sc_scatter_body)


def _node_body(u_ref, p0_ref, p1_ref, wn1ut_ref, wn1at_ref, bn1_ref,
               woutt_ref, y_ref):
    agg = p0_ref[...] + p1_ref[...]
    v = (jnp.dot(u_ref[...], wn1ut_ref[...], preferred_element_type=f32)
         + jnp.dot(agg, wn1at_ref[...], preferred_element_type=f32)
         + bn1_ref[...])
    y_ref[...] = jnp.dot(v, woutt_ref[...], preferred_element_type=f32)


def _node(u, p0, p1, wn1ut, wn1at, bn1, woutt):
    BN = 2000
    grid = (N // BN,)
    return pl.pallas_call(
        _node_body,
        grid=grid,
        in_specs=[
            pl.BlockSpec((BN, D), lambda i: (i, 0)),
            pl.BlockSpec((BN, D), lambda i: (i, 0)),
            pl.BlockSpec((BN, D), lambda i: (i, 0)),
            pl.BlockSpec((D, D), lambda i: (0, 0)),
            pl.BlockSpec((D, D), lambda i: (0, 0)),
            pl.BlockSpec((1, D), lambda i: (0, 0)),
            pl.BlockSpec((D, D), lambda i: (0, 0)),
        ],
        out_specs=pl.BlockSpec((BN, D), lambda i: (i, 0)),
        out_shape=jax.ShapeDtypeStruct((N, D), f32),
    )(u, p0, p1, wn1ut, wn1at, bn1, woutt)


def kernel(x, pos, edge_index, W_in, W_e1, b_e1, W_e2, b_e2, W_n1, b_n1, W_out):
    src = edge_index[0].reshape(NW, NCHUNK, C)
    dst = edge_index[1].reshape(NW, NCHUNK, C)
    posx = pos[:, 0]
    posy = pos[:, 1]
    posz = pos[:, 2]
    wd = W_e1[:, 2 * D].reshape(1, D)

    u, a, b = _prep(x, W_in.T, W_e1[:, :D].T, W_e1[:, D:2 * D].T,
                    b_e1.reshape(1, D))
    arows, brows, dist = _sc_gather(a, b, posx, posy, posz, src, dst)
    m2 = _edge_mlp(arows, brows, dist.reshape(E, 1), wd, W_e2.T,
                   b_e2.reshape(1, D))
    parts = _sc_scatter(m2, dst, jnp.zeros((NPAD, D), f32))
    y = _node(u, parts[0], parts[1], W_n1[:, :D].T, W_n1[:, D:].T,
              b_n1.reshape(1, D), W_out.T)
    return (y, pos)


# speedup vs baseline: 5.1021x; 1.0465x over previous
import functools

import jax
import jax.numpy as jnp
from jax import lax
from jax.experimental import pallas as pl
from jax.experimental.pallas import tpu as pltpu
from jax.experimental.pallas import tpu_sc as plsc

N = 10000
E = 320000
D = 128
NC = 2
NS = 16
NW = NC * NS
EPW = E // NW
C = 80
NCHUNK = EPW // C
NPAD = 10112
RPT = NPAD // NS

f32 = jnp.float32

_mesh = plsc.VectorSubcoreMesh(
    core_axis_name="c", subcore_axis_name="s", num_cores=NC, num_subcores=NS)
_sc_params = pltpu.CompilerParams(needs_layout_passes=False)


def _prep_body(x_ref, wint_ref, wdt_ref, wst_ref, be1_ref,
               u_ref, a_ref, b_ref):
    u = jnp.dot(x_ref[...], wint_ref[...], preferred_element_type=f32)
    u_ref[...] = u
    a_ref[...] = jnp.dot(u, wdt_ref[...], preferred_element_type=f32) + be1_ref[...]
    b_ref[...] = jnp.dot(u, wst_ref[...], preferred_element_type=f32)


def _prep(x, wint, wdt, wst, be1):
    BN = 2000
    grid = (N // BN,)
    return pl.pallas_call(
        _prep_body,
        grid=grid,
        in_specs=[
            pl.BlockSpec((BN, D), lambda i: (i, 0)),
            pl.BlockSpec((D, D), lambda i: (0, 0)),
            pl.BlockSpec((D, D), lambda i: (0, 0)),
            pl.BlockSpec((D, D), lambda i: (0, 0)),
            pl.BlockSpec((1, D), lambda i: (0, 0)),
        ],
        out_specs=[
            pl.BlockSpec((BN, D), lambda i: (i, 0)),
            pl.BlockSpec((BN, D), lambda i: (i, 0)),
            pl.BlockSpec((BN, D), lambda i: (i, 0)),
        ],
        out_shape=[
            jax.ShapeDtypeStruct((N, D), f32),
            jax.ShapeDtypeStruct((N, D), f32),
            jax.ShapeDtypeStruct((N, D), f32),
        ],
    )(x, wint, wdt, wst, be1)


def _sc_gather_body(a_hbm, b_hbm, posx_hbm, posy_hbm, posz_hbm,
                    srcg_hbm, dstg_hbm,
                    zrows_hbm, dist_hbm,
                    idxs_v, idxd_v, posx_v, posy_v, posz_v,
                    bufA0, bufB0, bufA1, bufB1, dist_v,
                    gA0, gB0, gA1, gB1, oA0, oB0, oA1, oB1):
    cid = lax.axis_index("c")
    sid = lax.axis_index("s")
    wid = sid * NC + cid
    pltpu.sync_copy(srcg_hbm.at[wid], idxs_v)
    pltpu.sync_copy(dstg_hbm.at[wid], idxd_v)
    pltpu.sync_copy(posx_hbm, posx_v)
    pltpu.sync_copy(posy_hbm, posy_v)
    pltpu.sync_copy(posz_hbm, posz_v)

    def issue_g(j, bA, bB, sA, sB):
        pltpu.async_copy(a_hbm.at[idxd_v.at[j]], bA, sA)
        pltpu.async_copy(b_hbm.at[idxs_v.at[j]], bB, sB)

    def wait_g(j, bA, bB, sA, sB):
        pltpu.make_async_copy(a_hbm.at[idxd_v.at[j]], bA, sA).wait()
        pltpu.make_async_copy(b_hbm.at[idxs_v.at[j]], bB, sB).wait()

    def issue_o(j, bA, bB, sA, sB):
        base = wid * EPW + j * C
        pltpu.async_copy(bA, zrows_hbm.at[pl.ds(base, C)], sA)

    def wait_o(j, bA, bB, sA, sB):
        base = wid * EPW + j * C
        pltpu.make_async_copy(bA, zrows_hbm.at[pl.ds(base, C)], sA).wait()

    def add_rows(bA, bB):
        def row_body(r, carry):
            for k in range(D // 16):
                sl = pl.ds(k * 16, 16)
                bA[r, sl] = bA[r, sl] + bB[r, sl]
            return carry
        lax.fori_loop(0, C, row_body, 0)

    def dist_chunk(j):
        for k in range(C // 16):
            s16 = idxs_v[j, pl.ds(k * 16, 16)]
            d16 = idxd_v[j, pl.ds(k * 16, 16)]
            dx = plsc.load_gather(posx_v, [s16]) - plsc.load_gather(posx_v, [d16])
            dy = plsc.load_gather(posy_v, [s16]) - plsc.load_gather(posy_v, [d16])
            dz = plsc.load_gather(posz_v, [s16]) - plsc.load_gather(posz_v, [d16])
            dist_v[0, pl.ds(j * C + k * 16, 16)] = dx * dx + dy * dy + dz * dz

    issue_g(0, bufA0, bufB0, gA0, gB0)
    issue_g(1, bufA1, bufB1, gA1, gB1)

    def body(i, carry):
        j0 = 2 * i
        j1 = j0 + 1
        n0 = jnp.minimum(j0 + 2, NCHUNK - 1)
        n1 = jnp.minimum(j1 + 2, NCHUNK - 1)
        dist_chunk(j0)
        wait_g(j0, bufA0, bufB0, gA0, gB0)
        add_rows(bufA0, bufB0)
        issue_o(j0, bufA0, bufB0, oA0, oB0)
        dist_chunk(j1)
        wait_g(j1, bufA1, bufB1, gA1, gB1)
        add_rows(bufA1, bufB1)
        issue_o(j1, bufA1, bufB1, oA1, oB1)
        wait_o(j0, bufA0, bufB0, oA0, oB0)
        issue_g(n0, bufA0, bufB0, gA0, gB0)
        wait_o(j1, bufA1, bufB1, oA1, oB1)
        issue_g(n1, bufA1, bufB1, gA1, gB1)
        return carry

    lax.fori_loop(0, (NCHUNK - 1) // 2, body, 0)
    j_last = NCHUNK - 1
    dist_chunk(j_last)
    wait_g(j_last, bufA0, bufB0, gA0, gB0)
    add_rows(bufA0, bufB0)
    issue_o(j_last, bufA0, bufB0, oA0, oB0)
    wait_g(j_last, bufA1, bufB1, gA1, gB1)
    wait_o(j_last, bufA0, bufB0, oA0, oB0)
    pltpu.sync_copy(dist_v, dist_hbm.at[wid])


_sc_gather = functools.partial(
    pl.kernel,
    out_type=(
        jax.ShapeDtypeStruct((E, D), f32),
        jax.ShapeDtypeStruct((NW, 1, EPW), f32),
    ),
    mesh=_mesh,
    compiler_params=_sc_params,
    scratch_types=[
        pltpu.VMEM((NCHUNK, C), jnp.int32),
        pltpu.VMEM((NCHUNK, C), jnp.int32),
        pltpu.VMEM((N,), f32),
        pltpu.VMEM((N,), f32),
        pltpu.VMEM((N,), f32),
        pltpu.VMEM((C, D), f32),
        pltpu.VMEM((C, D), f32),
        pltpu.VMEM((C, D), f32),
        pltpu.VMEM((C, D), f32),
        pltpu.VMEM((1, EPW), f32),
        pltpu.SemaphoreType.DMA,
        pltpu.SemaphoreType.DMA,
        pltpu.SemaphoreType.DMA,
        pltpu.SemaphoreType.DMA,
        pltpu.SemaphoreType.DMA,
        pltpu.SemaphoreType.DMA,
        pltpu.SemaphoreType.DMA,
        pltpu.SemaphoreType.DMA,
    ],
)(_sc_gather_body)


def _edge_body(z_ref, dist_ref, wd_ref, we2t_ref, be2_ref, out_ref):
    z = z_ref[...] + dist_ref[...] * wd_ref[...]
    z = z * jax.nn.sigmoid(z)
    m = jnp.dot(z, we2t_ref[...], preferred_element_type=f32) + be2_ref[...]
    out_ref[...] = m * jax.nn.sigmoid(m)


def _edge_mlp(zrows, dist, wd, we2t, be2):
    BE = 2000
    grid = (E // BE,)
    return pl.pallas_call(
        _edge_body,
        grid=grid,
        in_specs=[
            pl.BlockSpec((BE, D), lambda i: (i, 0)),
            pl.BlockSpec((BE, 1), lambda i: (i, 0)),
            pl.BlockSpec((1, D), lambda i: (0, 0)),
            pl.BlockSpec((D, D), lambda i: (0, 0)),
            pl.BlockSpec((1, D), lambda i: (0, 0)),
        ],
        out_specs=pl.BlockSpec((BE, D), lambda i: (i, 0)),
        out_shape=jax.ShapeDtypeStruct((E, D), f32),
    )(zrows, dist, wd, we2t, be2)


def _sc_scatter_body(m2_hbm, dstg_hbm, zeros_hbm, part_hbm,
                     idxd_v, buf0, buf1, agg_sh,
                     g0, g1, s0, s1):
    cid = lax.axis_index("c")
    sid = lax.axis_index("s")
    wid = sid * NC + cid
    pltpu.sync_copy(dstg_hbm.at[wid], idxd_v)
    pltpu.sync_copy(zeros_hbm.at[pl.ds(sid * RPT, RPT)],
                    agg_sh.at[pl.ds(sid * RPT, RPT)])
    plsc.subcore_barrier()

    def issue_g(j, b, s):
        pltpu.async_copy(m2_hbm.at[pl.ds(wid * EPW + j * C, C)], b, s)

    def wait_g(j, b, s):
        pltpu.make_async_copy(m2_hbm.at[pl.ds(wid * EPW + j * C, C)], b, s).wait()

    def issue_a(j, b, s):
        pltpu.async_copy(b, agg_sh.at[idxd_v.at[j]], s, add=True)

    def wait_a(j, b, s):
        pltpu.make_async_copy(b, agg_sh.at[idxd_v.at[j]], s).wait()

    issue_g(0, buf0, g0)
    issue_g(1, buf1, g1)

    def body(i, carry):
        j0 = 2 * i
        j1 = j0 + 1
        n0 = jnp.minimum(j0 + 2, NCHUNK - 1)
        n1 = jnp.minimum(j1 + 2, NCHUNK - 1)
        wait_g(j0, buf0, g0)
        issue_a(j0, buf0, s0)
        wait_g(j1, buf1, g1)
        issue_a(j1, buf1, s1)
        wait_a(j0, buf0, s0)
        issue_g(n0, buf0, g0)
        wait_a(j1, buf1, s1)
        issue_g(n1, buf1, g1)
        return carry

    lax.fori_loop(0, (NCHUNK - 1) // 2, body, 0)
    j_last = NCHUNK - 1
    wait_g(j_last, buf0, g0)
    issue_a(j_last, buf0, s0)
    wait_g(j_last, buf1, g1)
    wait_a(j_last, buf0, s0)
    plsc.subcore_barrier()
    pltpu.sync_copy(agg_sh.at[pl.ds(sid * RPT, RPT)],
                    part_hbm.at[cid, pl.ds(sid * RPT, RPT)])


_sc_scatter = functools.partial(
    pl.kernel,
    out_type=jax.ShapeDtypeStruct((NC, NPAD, D), f32),
    mesh=_mesh,
    compiler_params=_sc_params,
    scratch_types=[
        pltpu.VMEM((NCHUNK, C), jnp.int32),
        pltpu.VMEM((C, D), f32),
        pltpu.VMEM((C, D), f32),
        pltpu.VMEM_SHARED((NPAD, D), f32),
        pltpu.SemaphoreType.DMA,
        pltpu.SemaphoreType.DMA,
        pltpu.SemaphoreType.DMA,
        pltpu.SemaphoreType.DMA,
    ],
)(_sc_scatter_body)


def _node_body(u_ref, p0_ref, p1_ref, wn1ut_ref, wn1at_ref, bn1_ref,
               woutt_ref, y_ref):
    agg = p0_ref[...] + p1_ref[...]
    v = (jnp.dot(u_ref[...], wn1ut_ref[...], preferred_element_type=f32)
         + jnp.dot(agg, wn1at_ref[...], preferred_element_type=f32)
         + bn1_ref[...])
    y_ref[...] = jnp.dot(v, woutt_ref[...], preferred_element_type=f32)


def _node(u, p0, p1, wn1ut, wn1at, bn1, woutt):
    BN = 2000
    grid = (N // BN,)
    return pl.pallas_call(
        _node_body,
        grid=grid,
        in_specs=[
            pl.BlockSpec((BN, D), lambda i: (i, 0)),
            pl.BlockSpec((BN, D), lambda i: (i, 0)),
            pl.BlockSpec((BN, D), lambda i: (i, 0)),
            pl.BlockSpec((D, D), lambda i: (0, 0)),
            pl.BlockSpec((D, D), lambda i: (0, 0)),
            pl.BlockSpec((1, D), lambda i: (0, 0)),
            pl.BlockSpec((D, D), lambda i: (0, 0)),
        ],
        out_specs=pl.BlockSpec((BN, D), lambda i: (i, 0)),
        out_shape=jax.ShapeDtypeStruct((N, D), f32),
    )(u, p0, p1, wn1ut, wn1at, bn1, woutt)


def kernel(x, pos, edge_index, W_in, W_e1, b_e1, W_e2, b_e2, W_n1, b_n1, W_out):
    src = edge_index[0].reshape(NW, NCHUNK, C)
    dst = edge_index[1].reshape(NW, NCHUNK, C)
    posx = pos[:, 0]
    posy = pos[:, 1]
    posz = pos[:, 2]
    wd = W_e1[:, 2 * D].reshape(1, D)

    u, a, b = _prep(x, W_in.T, W_e1[:, :D].T, W_e1[:, D:2 * D].T,
                    b_e1.reshape(1, D))
    zrows, dist = _sc_gather(a, b, posx, posy, posz, src, dst)
    m2 = _edge_mlp(zrows, dist.reshape(E, 1), wd, W_e2.T,
                   b_e2.reshape(1, D))
    parts = _sc_scatter(m2, dst, jnp.zeros((NPAD, D), f32))
    y = _node(u, parts[0], parts[1], W_n1[:, :D].T, W_n1[:, D:].T,
              b_n1.reshape(1, D), W_out.T)
    return (y, pos)


# speedup vs baseline: 6.3649x; 1.2475x over previous
import functools

import jax
import jax.numpy as jnp
from jax import lax
from jax.experimental import pallas as pl
from jax.experimental.pallas import tpu as pltpu
from jax.experimental.pallas import tpu_sc as plsc

N = 10000
E = 320000
D = 128
NC = 2
NS = 16
NW = NC * NS
EPW = E // NW
C = 80
NCHUNK = EPW // C
NPAD = 10112
RPT = NPAD // NS

f32 = jnp.float32

_mesh = plsc.VectorSubcoreMesh(
    core_axis_name="c", subcore_axis_name="s", num_cores=NC, num_subcores=NS)
_sc_params = pltpu.CompilerParams(needs_layout_passes=False)


def _prep_body(x_ref, wint_ref, wdt_ref, wst_ref, be1_ref,
               u_ref, a_ref, b_ref):
    u = jnp.dot(x_ref[...], wint_ref[...], preferred_element_type=f32)
    u_ref[...] = u
    a_ref[...] = jnp.dot(u, wdt_ref[...], preferred_element_type=f32) + be1_ref[...]
    b_ref[...] = jnp.dot(u, wst_ref[...], preferred_element_type=f32)


def _prep(x, wint, wdt, wst, be1):
    BN = 2000
    grid = (N // BN,)
    return pl.pallas_call(
        _prep_body,
        grid=grid,
        in_specs=[
            pl.BlockSpec((BN, D), lambda i: (i, 0)),
            pl.BlockSpec((D, D), lambda i: (0, 0)),
            pl.BlockSpec((D, D), lambda i: (0, 0)),
            pl.BlockSpec((D, D), lambda i: (0, 0)),
            pl.BlockSpec((1, D), lambda i: (0, 0)),
        ],
        out_specs=[
            pl.BlockSpec((BN, D), lambda i: (i, 0)),
            pl.BlockSpec((BN, D), lambda i: (i, 0)),
            pl.BlockSpec((BN, D), lambda i: (i, 0)),
        ],
        out_shape=[
            jax.ShapeDtypeStruct((N, D), f32),
            jax.ShapeDtypeStruct((N, D), f32),
            jax.ShapeDtypeStruct((N, D), f32),
        ],
    )(x, wint, wdt, wst, be1)


def _sc_gather_body(a_hbm, b_hbm, posx_hbm, posy_hbm, posz_hbm,
                    srcg_hbm, dstg_hbm,
                    zrows_hbm, dist_hbm,
                    idxs_v, idxd_v, posx_v, posy_v, posz_v,
                    bufA0, bufB0, bufA1, bufB1, dist_v,
                    gA0, gB0, gA1, gB1, oA0, oB0, oA1, oB1):
    cid = lax.axis_index("c")
    sid = lax.axis_index("s")
    wid = sid * NC + cid
    pltpu.sync_copy(srcg_hbm.at[wid], idxs_v)
    pltpu.sync_copy(dstg_hbm.at[wid], idxd_v)
    pltpu.sync_copy(posx_hbm, posx_v)
    pltpu.sync_copy(posy_hbm, posy_v)
    pltpu.sync_copy(posz_hbm, posz_v)

    def issue_g(j, bA, bB, sA, sB):
        pltpu.async_copy(a_hbm.at[idxd_v.at[j]], bA, sA)
        pltpu.async_copy(b_hbm.at[idxs_v.at[j]], bB, sB)

    def wait_g(j, bA, bB, sA, sB):
        pltpu.make_async_copy(a_hbm.at[idxd_v.at[j]], bA, sA).wait()
        pltpu.make_async_copy(b_hbm.at[idxs_v.at[j]], bB, sB).wait()

    def issue_o(j, bA, bB, sA, sB):
        base = wid * EPW + j * C
        pltpu.async_copy(bA, zrows_hbm.at[pl.ds(base, C)], sA)

    def wait_o(j, bA, bB, sA, sB):
        base = wid * EPW + j * C
        pltpu.make_async_copy(bA, zrows_hbm.at[pl.ds(base, C)], sA).wait()

    def add_rows(bA, bB):
        def row_body(r, carry):
            for k in range(D // 16):
                sl = pl.ds(k * 16, 16)
                bA[r, sl] = bA[r, sl] + bB[r, sl]
            return carry
        lax.fori_loop(0, C, row_body, 0)

    def dist_chunk(j):
        for k in range(C // 16):
            s16 = idxs_v[j, pl.ds(k * 16, 16)]
            d16 = idxd_v[j, pl.ds(k * 16, 16)]
            dx = plsc.load_gather(posx_v, [s16]) - plsc.load_gather(posx_v, [d16])
            dy = plsc.load_gather(posy_v, [s16]) - plsc.load_gather(posy_v, [d16])
            dz = plsc.load_gather(posz_v, [s16]) - plsc.load_gather(posz_v, [d16])
            flat = j * C + k * 16
            dist_v[flat // 2000, 0, pl.ds(flat % 2000, 16)] = (
                dx * dx + dy * dy + dz * dz)

    issue_g(0, bufA0, bufB0, gA0, gB0)
    issue_g(1, bufA1, bufB1, gA1, gB1)

    def body(i, carry):
        j0 = 2 * i
        j1 = j0 + 1
        n0 = jnp.minimum(j0 + 2, NCHUNK - 1)
        n1 = jnp.minimum(j1 + 2, NCHUNK - 1)
        dist_chunk(j0)
        wait_g(j0, bufA0, bufB0, gA0, gB0)
        add_rows(bufA0, bufB0)
        issue_o(j0, bufA0, bufB0, oA0, oB0)
        dist_chunk(j1)
        wait_g(j1, bufA1, bufB1, gA1, gB1)
        add_rows(bufA1, bufB1)
        issue_o(j1, bufA1, bufB1, oA1, oB1)
        wait_o(j0, bufA0, bufB0, oA0, oB0)
        issue_g(n0, bufA0, bufB0, gA0, gB0)
        wait_o(j1, bufA1, bufB1, oA1, oB1)
        issue_g(n1, bufA1, bufB1, gA1, gB1)
        return carry

    lax.fori_loop(0, (NCHUNK - 1) // 2, body, 0)
    j_last = NCHUNK - 1
    dist_chunk(j_last)
    wait_g(j_last, bufA0, bufB0, gA0, gB0)
    add_rows(bufA0, bufB0)
    issue_o(j_last, bufA0, bufB0, oA0, oB0)
    wait_g(j_last, bufA1, bufB1, gA1, gB1)
    wait_o(j_last, bufA0, bufB0, oA0, oB0)
    pltpu.sync_copy(dist_v, dist_hbm.at[pl.ds(wid * (EPW // 2000), EPW // 2000)])


_sc_gather = functools.partial(
    pl.kernel,
    out_type=(
        jax.ShapeDtypeStruct((E, D), f32),
        jax.ShapeDtypeStruct((NW * (EPW // 2000), 1, 2000), f32),
    ),
    mesh=_mesh,
    compiler_params=_sc_params,
    scratch_types=[
        pltpu.VMEM((NCHUNK, C), jnp.int32),
        pltpu.VMEM((NCHUNK, C), jnp.int32),
        pltpu.VMEM((N,), f32),
        pltpu.VMEM((N,), f32),
        pltpu.VMEM((N,), f32),
        pltpu.VMEM((C, D), f32),
        pltpu.VMEM((C, D), f32),
        pltpu.VMEM((C, D), f32),
        pltpu.VMEM((C, D), f32),
        pltpu.VMEM((EPW // 2000, 1, 2000), f32),
        pltpu.SemaphoreType.DMA,
        pltpu.SemaphoreType.DMA,
        pltpu.SemaphoreType.DMA,
        pltpu.SemaphoreType.DMA,
        pltpu.SemaphoreType.DMA,
        pltpu.SemaphoreType.DMA,
        pltpu.SemaphoreType.DMA,
        pltpu.SemaphoreType.DMA,
    ],
)(_sc_gather_body)


def _edge_body(z_ref, dist_ref, wd_ref, we2t_ref, be2_ref, out_ref):
    de = dist_ref[0]
    dist_term = lax.dot_general(de, wd_ref[...], (((0,), (0,)), ((), ())),
                                preferred_element_type=f32)
    z = z_ref[...] + dist_term
    z = z * jax.nn.sigmoid(z)
    m = jnp.dot(z, we2t_ref[...], preferred_element_type=f32) + be2_ref[...]
    out_ref[...] = m * jax.nn.sigmoid(m)


def _edge_mlp(zrows, dist, wd, we2t, be2):
    BE = 2000
    grid = (E // BE,)
    return pl.pallas_call(
        _edge_body,
        grid=grid,
        in_specs=[
            pl.BlockSpec((BE, D), lambda i: (i, 0)),
            pl.BlockSpec((1, 1, BE), lambda i: (i, 0, 0)),
            pl.BlockSpec((1, D), lambda i: (0, 0)),
            pl.BlockSpec((D, D), lambda i: (0, 0)),
            pl.BlockSpec((1, D), lambda i: (0, 0)),
        ],
        out_specs=pl.BlockSpec((BE, D), lambda i: (i, 0)),
        out_shape=jax.ShapeDtypeStruct((E, D), f32),
    )(zrows, dist, wd, we2t, be2)


def _sc_scatter_body(m2_hbm, dstg_hbm, zeros_hbm, part_hbm,
                     idxd_v, buf0, buf1, agg_sh,
                     g0, g1, s0, s1):
    cid = lax.axis_index("c")
    sid = lax.axis_index("s")
    wid = sid * NC + cid
    pltpu.sync_copy(dstg_hbm.at[wid], idxd_v)
    pltpu.sync_copy(zeros_hbm.at[pl.ds(sid * RPT, RPT)],
                    agg_sh.at[pl.ds(sid * RPT, RPT)])
    plsc.subcore_barrier()

    def issue_g(j, b, s):
        pltpu.async_copy(m2_hbm.at[pl.ds(wid * EPW + j * C, C)], b, s)

    def wait_g(j, b, s):
        pltpu.make_async_copy(m2_hbm.at[pl.ds(wid * EPW + j * C, C)], b, s).wait()

    def issue_a(j, b, s):
        pltpu.async_copy(b, agg_sh.at[idxd_v.at[j]], s, add=True)

    def wait_a(j, b, s):
        pltpu.make_async_copy(b, agg_sh.at[idxd_v.at[j]], s).wait()

    issue_g(0, buf0, g0)
    issue_g(1, buf1, g1)

    def body(i, carry):
        j0 = 2 * i
        j1 = j0 + 1
        n0 = jnp.minimum(j0 + 2, NCHUNK - 1)
        n1 = jnp.minimum(j1 + 2, NCHUNK - 1)
        wait_g(j0, buf0, g0)
        issue_a(j0, buf0, s0)
        wait_g(j1, buf1, g1)
        issue_a(j1, buf1, s1)
        wait_a(j0, buf0, s0)
        issue_g(n0, buf0, g0)
        wait_a(j1, buf1, s1)
        issue_g(n1, buf1, g1)
        return carry

    lax.fori_loop(0, (NCHUNK - 1) // 2, body, 0)
    j_last = NCHUNK - 1
    wait_g(j_last, buf0, g0)
    issue_a(j_last, buf0, s0)
    wait_g(j_last, buf1, g1)
    wait_a(j_last, buf0, s0)
    plsc.subcore_barrier()
    pltpu.sync_copy(agg_sh.at[pl.ds(sid * RPT, RPT)],
                    part_hbm.at[cid, pl.ds(sid * RPT, RPT)])


_sc_scatter = functools.partial(
    pl.kernel,
    out_type=jax.ShapeDtypeStruct((NC, NPAD, D), f32),
    mesh=_mesh,
    compiler_params=_sc_params,
    scratch_types=[
        pltpu.VMEM((NCHUNK, C), jnp.int32),
        pltpu.VMEM((C, D), f32),
        pltpu.VMEM((C, D), f32),
        pltpu.VMEM_SHARED((NPAD, D), f32),
        pltpu.SemaphoreType.DMA,
        pltpu.SemaphoreType.DMA,
        pltpu.SemaphoreType.DMA,
        pltpu.SemaphoreType.DMA,
    ],
)(_sc_scatter_body)


def _node_body(u_ref, p0_ref, p1_ref, wn1ut_ref, wn1at_ref, bn1_ref,
               woutt_ref, y_ref):
    agg = p0_ref[...] + p1_ref[...]
    v = (jnp.dot(u_ref[...], wn1ut_ref[...], preferred_element_type=f32)
         + jnp.dot(agg, wn1at_ref[...], preferred_element_type=f32)
         + bn1_ref[...])
    y_ref[...] = jnp.dot(v, woutt_ref[...], preferred_element_type=f32)


def _node(u, p0, p1, wn1ut, wn1at, bn1, woutt):
    BN = 2000
    grid = (N // BN,)
    return pl.pallas_call(
        _node_body,
        grid=grid,
        in_specs=[
            pl.BlockSpec((BN, D), lambda i: (i, 0)),
            pl.BlockSpec((BN, D), lambda i: (i, 0)),
            pl.BlockSpec((BN, D), lambda i: (i, 0)),
            pl.BlockSpec((D, D), lambda i: (0, 0)),
            pl.BlockSpec((D, D), lambda i: (0, 0)),
            pl.BlockSpec((1, D), lambda i: (0, 0)),
            pl.BlockSpec((D, D), lambda i: (0, 0)),
        ],
        out_specs=pl.BlockSpec((BN, D), lambda i: (i, 0)),
        out_shape=jax.ShapeDtypeStruct((N, D), f32),
    )(u, p0, p1, wn1ut, wn1at, bn1, woutt)


def kernel(x, pos, edge_index, W_in, W_e1, b_e1, W_e2, b_e2, W_n1, b_n1, W_out):
    src = edge_index[0].reshape(NW, NCHUNK, C)
    dst = edge_index[1].reshape(NW, NCHUNK, C)
    posx = pos[:, 0]
    posy = pos[:, 1]
    posz = pos[:, 2]
    wd = W_e1[:, 2 * D].reshape(1, D)

    u, a, b = _prep(x, W_in.T, W_e1[:, :D].T, W_e1[:, D:2 * D].T,
                    b_e1.reshape(1, D))
    zrows, dist = _sc_gather(a, b, posx, posy, posz, src, dst)
    m2 = _edge_mlp(zrows, dist, wd, W_e2.T, b_e2.reshape(1, D))
    parts = _sc_scatter(m2, dst, jnp.zeros((NPAD, D), f32))
    y = _node(u, parts[0], parts[1], W_n1[:, :D].T, W_n1[:, D:].T,
              b_n1.reshape(1, D), W_out.T)
    return (y, pos)


# speedup vs baseline: 7.8070x; 1.2266x over previous
import functools

import jax
import jax.numpy as jnp
from jax import lax
from jax.experimental import pallas as pl
from jax.experimental.pallas import tpu as pltpu
from jax.experimental.pallas import tpu_sc as plsc

N = 10000
E = 320000
D = 128
NC = 2
NS = 16
NW = NC * NS
C = 80
NPAD = 10112
RPT = NPAD // NS
DROW = 2000
E1 = 192000
E2 = E - E1
BE = 2000

f32 = jnp.float32

_mesh = plsc.VectorSubcoreMesh(
    core_axis_name="c", subcore_axis_name="s", num_cores=NC, num_subcores=NS)
_sc_params = pltpu.CompilerParams(needs_layout_passes=False)


def _prep_body(x_ref, wint_ref, wdt_ref, wst_ref, be1_ref,
               u_ref, a_ref, b_ref):
    u = jnp.dot(x_ref[...], wint_ref[...], preferred_element_type=f32)
    u_ref[...] = u
    a_ref[...] = jnp.dot(u, wdt_ref[...], preferred_element_type=f32) + be1_ref[...]
    b_ref[...] = jnp.dot(u, wst_ref[...], preferred_element_type=f32)


def _prep(x, wint, wdt, wst, be1):
    BN = 2000
    grid = (N // BN,)
    return pl.pallas_call(
        _prep_body,
        grid=grid,
        in_specs=[
            pl.BlockSpec((BN, D), lambda i: (i, 0)),
            pl.BlockSpec((D, D), lambda i: (0, 0)),
            pl.BlockSpec((D, D), lambda i: (0, 0)),
            pl.BlockSpec((D, D), lambda i: (0, 0)),
            pl.BlockSpec((1, D), lambda i: (0, 0)),
        ],
        out_specs=[
            pl.BlockSpec((BN, D), lambda i: (i, 0)),
            pl.BlockSpec((BN, D), lambda i: (i, 0)),
            pl.BlockSpec((BN, D), lambda i: (i, 0)),
        ],
        out_shape=[
            jax.ShapeDtypeStruct((N, D), f32),
            jax.ShapeDtypeStruct((N, D), f32),
            jax.ShapeDtypeStruct((N, D), f32),
        ],
    )(x, wint, wdt, wst, be1)


def _make_sc_gather(eh):
    epw = eh // NW
    nchunk = epw // C
    drpw = epw // DROW

    def body(a_hbm, b_hbm, posx_hbm, posy_hbm, posz_hbm,
             srcg_hbm, dstg_hbm,
             zrows_hbm, dist_hbm,
             idxs_v, idxd_v, posx_v, posy_v, posz_v,
             bufA0, bufB0, bufA1, bufB1, dist_v,
             gA0, gB0, gA1, gB1, oA0, oB0, oA1, oB1):
        cid = lax.axis_index("c")
        sid = lax.axis_index("s")
        wid = sid * NC + cid
        pltpu.sync_copy(srcg_hbm.at[wid], idxs_v)
        pltpu.sync_copy(dstg_hbm.at[wid], idxd_v)
        pltpu.sync_copy(posx_hbm, posx_v)
        pltpu.sync_copy(posy_hbm, posy_v)
        pltpu.sync_copy(posz_hbm, posz_v)

        def issue_g(j, bA, bB, sA, sB):
            pltpu.async_copy(a_hbm.at[idxd_v.at[j]], bA, sA)
            pltpu.async_copy(b_hbm.at[idxs_v.at[j]], bB, sB)

        def wait_g(j, bA, bB, sA, sB):
            pltpu.make_async_copy(a_hbm.at[idxd_v.at[j]], bA, sA).wait()
            pltpu.make_async_copy(b_hbm.at[idxs_v.at[j]], bB, sB).wait()

        def issue_o(j, bA, sA):
            base = wid * epw + j * C
            pltpu.async_copy(bA, zrows_hbm.at[pl.ds(base, C)], sA)

        def wait_o(j, bA, sA):
            base = wid * epw + j * C
            pltpu.make_async_copy(bA, zrows_hbm.at[pl.ds(base, C)], sA).wait()

        def add_rows(bA, bB):
            def row_body(r, carry):
                for k in range(D // 16):
                    sl = pl.ds(k * 16, 16)
                    bA[r, sl] = bA[r, sl] + bB[r, sl]
                return carry
            lax.fori_loop(0, C, row_body, 0)

        def dist_chunk(j):
            for k in range(C // 16):
                s16 = idxs_v[j, pl.ds(k * 16, 16)]
                d16 = idxd_v[j, pl.ds(k * 16, 16)]
                dx = plsc.load_gather(posx_v, [s16]) - plsc.load_gather(posx_v, [d16])
                dy = plsc.load_gather(posy_v, [s16]) - plsc.load_gather(posy_v, [d16])
                dz = plsc.load_gather(posz_v, [s16]) - plsc.load_gather(posz_v, [d16])
                flat = j * C + k * 16
                dist_v[flat // DROW, 0, pl.ds(flat % DROW, 16)] = (
                    dx * dx + dy * dy + dz * dz)

        def process(j, bA, bB, gA, gB, oA):
            dist_chunk(j)
            wait_g(j, bA, bB, gA, gB)
            add_rows(bA, bB)
            issue_o(j, bA, oA)

        issue_g(0, bufA0, bufB0, gA0, gB0)
        issue_g(1, bufA1, bufB1, gA1, gB1)

        def loop(i, carry):
            j0 = 2 * i
            j1 = j0 + 1
            n0 = jnp.minimum(j0 + 2, nchunk - 1)
            n1 = jnp.minimum(j1 + 2, nchunk - 1)
            process(j0, bufA0, bufB0, gA0, gB0, oA0)
            process(j1, bufA1, bufB1, gA1, gB1, oA1)
            wait_o(j0, bufA0, oA0)
            issue_g(n0, bufA0, bufB0, gA0, gB0)
            wait_o(j1, bufA1, oA1)
            issue_g(n1, bufA1, bufB1, gA1, gB1)
            return carry

        lax.fori_loop(0, (nchunk - 1) // 2, loop, 0)
        if nchunk % 2 == 1:
            j = nchunk - 1
            process(j, bufA0, bufB0, gA0, gB0, oA0)
            wait_g(j, bufA1, bufB1, gA1, gB1)
            wait_o(j, bufA0, oA0)
        else:
            j0 = nchunk - 2
            j1 = nchunk - 1
            process(j0, bufA0, bufB0, gA0, gB0, oA0)
            process(j1, bufA1, bufB1, gA1, gB1, oA1)
            wait_o(j0, bufA0, oA0)
            wait_o(j1, bufA1, oA1)
        pltpu.sync_copy(dist_v, dist_hbm.at[pl.ds(wid * drpw, drpw)])

    return functools.partial(
        pl.kernel,
        out_type=(
            jax.ShapeDtypeStruct((eh, D), f32),
            jax.ShapeDtypeStruct((eh // DROW, 1, DROW), f32),
        ),
        mesh=_mesh,
        compiler_params=_sc_params,
        scratch_types=[
            pltpu.VMEM((nchunk, C), jnp.int32),
            pltpu.VMEM((nchunk, C), jnp.int32),
            pltpu.VMEM((N,), f32),
            pltpu.VMEM((N,), f32),
            pltpu.VMEM((N,), f32),
            pltpu.VMEM((C, D), f32),
            pltpu.VMEM((C, D), f32),
            pltpu.VMEM((C, D), f32),
            pltpu.VMEM((C, D), f32),
            pltpu.VMEM((drpw, 1, DROW), f32),
        ] + [pltpu.SemaphoreType.DMA] * 8,
    )(body)


def _edge_body(z_ref, dist_ref, wd_ref, we2t_ref, be2_ref, out_ref):
    de = dist_ref[0]
    dist_term = lax.dot_general(de, wd_ref[...], (((0,), (0,)), ((), ())),
                                preferred_element_type=f32)
    z = z_ref[...] + dist_term
    z = z * jax.nn.sigmoid(z)
    m = jnp.dot(z, we2t_ref[...], preferred_element_type=f32) + be2_ref[...]
    out_ref[...] = m * jax.nn.sigmoid(m)


def _edge_mlp(zrows, dist, wd, we2t, be2, eh):
    grid = (eh // BE,)
    return pl.pallas_call(
        _edge_body,
        grid=grid,
        in_specs=[
            pl.BlockSpec((BE, D), lambda i: (i, 0)),
            pl.BlockSpec((1, 1, BE), lambda i: (i, 0, 0)),
            pl.BlockSpec((1, D), lambda i: (0, 0)),
            pl.BlockSpec((D, D), lambda i: (0, 0)),
            pl.BlockSpec((1, D), lambda i: (0, 0)),
        ],
        out_specs=pl.BlockSpec((BE, D), lambda i: (i, 0)),
        out_shape=jax.ShapeDtypeStruct((eh, D), f32),
    )(zrows, dist, wd, we2t, be2)


def _make_sc_scatter(eh):
    epw = eh // NW
    nchunk = epw // C

    def body(m2_hbm, dstg_hbm, zeros_hbm, part_hbm,
             idxd_v, buf0, buf1, agg_sh,
             g0, g1, s0, s1):
        cid = lax.axis_index("c")
        sid = lax.axis_index("s")
        wid = sid * NC + cid
        pltpu.sync_copy(dstg_hbm.at[wid], idxd_v)
        pltpu.sync_copy(zeros_hbm.at[pl.ds(sid * RPT, RPT)],
                        agg_sh.at[pl.ds(sid * RPT, RPT)])
        plsc.subcore_barrier()

        def issue_g(j, b, s):
            pltpu.async_copy(m2_hbm.at[pl.ds(wid * epw + j * C, C)], b, s)

        def wait_g(j, b, s):
            pltpu.make_async_copy(
                m2_hbm.at[pl.ds(wid * epw + j * C, C)], b, s).wait()

        def issue_a(j, b, s):
            pltpu.async_copy(b, agg_sh.at[idxd_v.at[j]], s, add=True)

        def wait_a(j, b, s):
            pltpu.make_async_copy(b, agg_sh.at[idxd_v.at[j]], s).wait()

        issue_g(0, buf0, g0)
        issue_g(1, buf1, g1)

        def loop(i, carry):
            j0 = 2 * i
            j1 = j0 + 1
            n0 = jnp.minimum(j0 + 2, nchunk - 1)
            n1 = jnp.minimum(j1 + 2, nchunk - 1)
            wait_g(j0, buf0, g0)
            issue_a(j0, buf0, s0)
            wait_g(j1, buf1, g1)
            issue_a(j1, buf1, s1)
            wait_a(j0, buf0, s0)
            issue_g(n0, buf0, g0)
            wait_a(j1, buf1, s1)
            issue_g(n1, buf1, g1)
            return carry

        lax.fori_loop(0, (nchunk - 1) // 2, loop, 0)
        if nchunk % 2 == 1:
            j = nchunk - 1
            wait_g(j, buf0, g0)
            issue_a(j, buf0, s0)
            wait_g(j, buf1, g1)
            wait_a(j, buf0, s0)
        else:
            j0 = nchunk - 2
            j1 = nchunk - 1
            wait_g(j0, buf0, g0)
            issue_a(j0, buf0, s0)
            wait_g(j1, buf1, g1)
            issue_a(j1, buf1, s1)
            wait_a(j0, buf0, s0)
            wait_a(j1, buf1, s1)
        plsc.subcore_barrier()
        pltpu.sync_copy(agg_sh.at[pl.ds(sid * RPT, RPT)],
                        part_hbm.at[cid, pl.ds(sid * RPT, RPT)])

    return functools.partial(
        pl.kernel,
        out_type=jax.ShapeDtypeStruct((NC, NPAD, D), f32),
        mesh=_mesh,
        compiler_params=_sc_params,
        scratch_types=[
            pltpu.VMEM((nchunk, C), jnp.int32),
            pltpu.VMEM((C, D), f32),
            pltpu.VMEM((C, D), f32),
            pltpu.VMEM_SHARED((NPAD, D), f32),
        ] + [pltpu.SemaphoreType.DMA] * 4,
    )(body)


_sc_gather1 = _make_sc_gather(E1)
_sc_gather2 = _make_sc_gather(E2)
_sc_scatter1 = _make_sc_scatter(E1)
_sc_scatter2 = _make_sc_scatter(E2)


def _node_body(u_ref, p1_ref, p2_ref, wn1ut_ref, wn1at_ref, bn1_ref,
               woutt_ref, y_ref):
    agg = (p1_ref[0] + p1_ref[1]) + (p2_ref[0] + p2_ref[1])
    v = (jnp.dot(u_ref[...], wn1ut_ref[...], preferred_element_type=f32)
         + jnp.dot(agg, wn1at_ref[...], preferred_element_type=f32)
         + bn1_ref[...])
    y_ref[...] = jnp.dot(v, woutt_ref[...], preferred_element_type=f32)


def _node(u, p1, p2, wn1ut, wn1at, bn1, woutt):
    BN = 2000
    grid = (N // BN,)
    return pl.pallas_call(
        _node_body,
        grid=grid,
        in_specs=[
            pl.BlockSpec((BN, D), lambda i: (i, 0)),
            pl.BlockSpec((NC, BN, D), lambda i: (0, i, 0)),
            pl.BlockSpec((NC, BN, D), lambda i: (0, i, 0)),
            pl.BlockSpec((D, D), lambda i: (0, 0)),
            pl.BlockSpec((D, D), lambda i: (0, 0)),
            pl.BlockSpec((1, D), lambda i: (0, 0)),
            pl.BlockSpec((D, D), lambda i: (0, 0)),
        ],
        out_specs=pl.BlockSpec((BN, D), lambda i: (i, 0)),
        out_shape=jax.ShapeDtypeStruct((N, D), f32),
    )(u, p1, p2, wn1ut, wn1at, bn1, woutt)


def kernel(x, pos, edge_index, W_in, W_e1, b_e1, W_e2, b_e2, W_n1, b_n1, W_out):
    src1 = edge_index[0, :E1].reshape(NW, E1 // NW // C, C)
    dst1 = edge_index[1, :E1].reshape(NW, E1 // NW // C, C)
    src2 = edge_index[0, E1:].reshape(NW, E2 // NW // C, C)
    dst2 = edge_index[1, E1:].reshape(NW, E2 // NW // C, C)
    posx = pos[:, 0]
    posy = pos[:, 1]
    posz = pos[:, 2]
    wd = W_e1[:, 2 * D].reshape(1, D)
    we2t = W_e2.T
    be2 = b_e2.reshape(1, D)

    u, a, b = _prep(x, W_in.T, W_e1[:, :D].T, W_e1[:, D:2 * D].T,
                    b_e1.reshape(1, D))
    z1, d1 = _sc_gather1(a, b, posx, posy, posz, src1, dst1)
    z2, d2 = _sc_gather2(a, b, posx, posy, posz, src2, dst2)
    m1 = _edge_mlp(z1, d1, wd, we2t, be2, E1)
    m2 = _edge_mlp(z2, d2, wd, we2t, be2, E2)
    zeros = jnp.zeros((NPAD, D), f32)
    p1 = _sc_scatter1(m1, dst1, zeros)
    p2 = _sc_scatter2(m2, dst2, zeros)
    y = _node(u, p1, p2, W_n1[:, :D].T, W_n1[:, D:].T,
              b_n1.reshape(1, D), W_out.T)
    return (y, pos)
